# baseline (speedup 1.0000x reference)
import functools

import jax
import jax.numpy as jnp
from jax import lax
from jax.experimental import pallas as pl
from jax.experimental.pallas import tpu as pltpu
from jax.experimental.pallas import tpu_sc as plsc

_BLOCK_SIZE = 8192
_N_EMBD = 1024

_info = plsc.get_sparse_core_info()
_NC, _NS = _info.num_cores, _info.num_subcores
_NW = _NC * _NS
_ROWS_PER_W = _BLOCK_SIZE // _NW

_CHUNK = 32
_NBUF = 2
_NCHUNKS = _ROWS_PER_W // _CHUNK


def _sc_copy(wpe_hbm, out_hbm, buf, *sems):
    sin = sems[:_NBUF]
    sout = sems[_NBUF:]
    wid = lax.axis_index("c") * _NS + lax.axis_index("s")
    base = wid * _ROWS_PER_W

    def cin(i):
        b = i % _NBUF
        return pltpu.async_copy(
            wpe_hbm.at[pl.ds(base + i * _CHUNK, _CHUNK)], buf.at[b], sin[b]
        )

    def cout(i):
        b = i % _NBUF
        return pltpu.async_copy(
            buf.at[b], out_hbm.at[pl.ds(base + i * _CHUNK, _CHUNK)], sout[b]
        )

    ins = [None] * _NCHUNKS
    outs = [None] * _NCHUNKS
    for i in range(_NCHUNKS):
        if i >= _NBUF:
            outs[i - _NBUF].wait()
        ins[i] = cin(i)
        if i >= 1:
            ins[i - 1].wait()
            outs[i - 1] = cout(i - 1)
    ins[_NCHUNKS - 1].wait()
    outs[_NCHUNKS - 1] = cout(_NCHUNKS - 1)
    for j in range(_NCHUNKS - _NBUF, _NCHUNKS):
        outs[j].wait()


def kernel(wpe):
    mesh = plsc.VectorSubcoreMesh(core_axis_name="c", subcore_axis_name="s")
    run = functools.partial(
        pl.kernel,
        mesh=mesh,
        out_type=jax.ShapeDtypeStruct((_BLOCK_SIZE, _N_EMBD), jnp.float32),
        scratch_types=(
            [pltpu.VMEM((_NBUF, _CHUNK, _N_EMBD), jnp.float32)]
            + [pltpu.SemaphoreType.DMA] * (2 * _NBUF)
        ),
    )(_sc_copy)
    return run(wpe)[None]


# speedup vs baseline: 1.0325x; 1.0325x over previous
import functools

import jax
import jax.numpy as jnp
from jax import lax
from jax.experimental import pallas as pl
from jax.experimental.pallas import tpu as pltpu
from jax.experimental.pallas import tpu_sc as plsc

_BLOCK_SIZE = 8192
_N_EMBD = 1024

_info = plsc.get_sparse_core_info()
_NC, _NS = _info.num_cores, _info.num_subcores
_NW = _NC * _NS
_ROWS_PER_W = _BLOCK_SIZE // _NW

_CHUNK = 32
_NBUF_A = 2
_NBUF_B = 2
_NCHUNKS = _ROWS_PER_W // _CHUNK
_A_CHUNKS = (0, 2, 4, 6, 7)
_B_CHUNKS = (1, 3, 5)


class _Ring:
    def __init__(self, in_fn, out_fn, n, nbuf):
        self.in_fn, self.out_fn, self.n, self.nbuf = in_fn, out_fn, n, nbuf
        self.ins = [None] * n
        self.outs = [None] * n

    def start(self, i):
        if i >= self.nbuf:
            self.outs[i - self.nbuf].wait()
        self.ins[i] = self.in_fn(i)

    def flush(self, i):
        if i >= 1:
            self.ins[i - 1].wait()
            self.outs[i - 1] = self.out_fn(i - 1)

    def finish(self):
        self.ins[self.n - 1].wait()
        self.outs[self.n - 1] = self.out_fn(self.n - 1)
        for j in range(max(0, self.n - self.nbuf), self.n):
            self.outs[j].wait()


def _sc_copy(wpe_hbm, out_hbm, vbuf, sbuf, *sems):
    via = sems[0:_NBUF_A]
    voa = sems[_NBUF_A : 2 * _NBUF_A]
    sib = sems[2 * _NBUF_A : 2 * _NBUF_A + _NBUF_B]
    sob = sems[2 * _NBUF_A + _NBUF_B :]
    cid = lax.axis_index("c")
    sid = lax.axis_index("s")
    base = (cid * _NS + sid) * _ROWS_PER_W

    def rows(g):
        return pl.ds(base + g * _CHUNK, _CHUNK)

    def a_in(i):
        return pltpu.async_copy(
            wpe_hbm.at[rows(_A_CHUNKS[i])], vbuf.at[i % _NBUF_A], via[i % _NBUF_A]
        )

    def a_out(i):
        return pltpu.async_copy(
            vbuf.at[i % _NBUF_A], out_hbm.at[rows(_A_CHUNKS[i])], voa[i % _NBUF_A]
        )

    def b_in(i):
        return pltpu.async_copy(
            wpe_hbm.at[rows(_B_CHUNKS[i])], sbuf.at[sid, i % _NBUF_B], sib[i % _NBUF_B]
        )

    def b_out(i):
        return pltpu.async_copy(
            sbuf.at[sid, i % _NBUF_B], out_hbm.at[rows(_B_CHUNKS[i])], sob[i % _NBUF_B]
        )

    ra = _Ring(a_in, a_out, len(_A_CHUNKS), _NBUF_A)
    rb = _Ring(b_in, b_out, len(_B_CHUNKS), _NBUF_B)
    for i in range(max(len(_A_CHUNKS), len(_B_CHUNKS))):
        if i < len(_A_CHUNKS):
            ra.start(i)
        if i < len(_B_CHUNKS):
            rb.start(i)
        if i < len(_A_CHUNKS):
            ra.flush(i)
        if i < len(_B_CHUNKS):
            rb.flush(i)
    ra.finish()
    rb.finish()


def kernel(wpe):
    mesh = plsc.VectorSubcoreMesh(core_axis_name="c", subcore_axis_name="s")
    run = functools.partial(
        pl.kernel,
        mesh=mesh,
        out_type=jax.ShapeDtypeStruct((_BLOCK_SIZE, _N_EMBD), jnp.float32),
        scratch_types=(
            [
                pltpu.VMEM((_NBUF_A, _CHUNK, _N_EMBD), jnp.float32),
                pltpu.VMEM_SHARED((_NS, _NBUF_B, _CHUNK, _N_EMBD), jnp.float32),
            ]
            + [pltpu.SemaphoreType.DMA] * (2 * _NBUF_A + 2 * _NBUF_B)
        ),
    )(_sc_copy)
    return run(wpe)[None]


# speedup vs baseline: 1.0328x; 1.0003x over previous
import functools

import jax
import jax.numpy as jnp
from jax import lax
from jax.experimental import pallas as pl
from jax.experimental.pallas import tpu as pltpu
from jax.experimental.pallas import tpu_sc as plsc

_BLOCK_SIZE = 8192
_N_EMBD = 1024

_info = plsc.get_sparse_core_info()
_NC, _NS = _info.num_cores, _info.num_subcores
_NW = _NC * _NS
_ROWS_PER_W = _BLOCK_SIZE // _NW

_CHUNK = 32
_NBUF_A = 2
_NBUF_B = 2
_NCHUNKS = _ROWS_PER_W // _CHUNK
_A_CHUNKS = (0, 2, 4)
_B_CHUNKS = (1, 3, 5, 6, 7)


class _Ring:
    def __init__(self, in_fn, out_fn, n, nbuf):
        self.in_fn, self.out_fn, self.n, self.nbuf = in_fn, out_fn, n, nbuf
        self.ins = [None] * n
        self.outs = [None] * n

    def start(self, i):
        if i >= self.nbuf:
            self.outs[i - self.nbuf].wait()
        self.ins[i] = self.in_fn(i)

    def flush(self, i):
        if i >= 1:
            self.ins[i - 1].wait()
            self.outs[i - 1] = self.out_fn(i - 1)

    def finish(self):
        self.ins[self.n - 1].wait()
        self.outs[self.n - 1] = self.out_fn(self.n - 1)
        for j in range(max(0, self.n - self.nbuf), self.n):
            self.outs[j].wait()


def _sc_copy(wpe_hbm, out_hbm, vbuf, sbuf, *sems):
    via = sems[0:_NBUF_A]
    voa = sems[_NBUF_A : 2 * _NBUF_A]
    sib = sems[2 * _NBUF_A : 2 * _NBUF_A + _NBUF_B]
    sob = sems[2 * _NBUF_A + _NBUF_B :]
    cid = lax.axis_index("c")
    sid = lax.axis_index("s")
    base = (cid * _NS + sid) * _ROWS_PER_W

    def rows(g):
        return pl.ds(base + g * _CHUNK, _CHUNK)

    def a_in(i):
        return pltpu.async_copy(
            wpe_hbm.at[rows(_A_CHUNKS[i])], vbuf.at[i % _NBUF_A], via[i % _NBUF_A]
        )

    def a_out(i):
        return pltpu.async_copy(
            vbuf.at[i % _NBUF_A], out_hbm.at[rows(_A_CHUNKS[i])], voa[i % _NBUF_A]
        )

    def b_in(i):
        return pltpu.async_copy(
            wpe_hbm.at[rows(_B_CHUNKS[i])], sbuf.at[sid, i % _NBUF_B], sib[i % _NBUF_B]
        )

    def b_out(i):
        return pltpu.async_copy(
            sbuf.at[sid, i % _NBUF_B], out_hbm.at[rows(_B_CHUNKS[i])], sob[i % _NBUF_B]
        )

    ra = _Ring(a_in, a_out, len(_A_CHUNKS), _NBUF_A)
    rb = _Ring(b_in, b_out, len(_B_CHUNKS), _NBUF_B)
    for i in range(max(len(_A_CHUNKS), len(_B_CHUNKS))):
        if i < len(_A_CHUNKS):
            ra.start(i)
        if i < len(_B_CHUNKS):
            rb.start(i)
        if i < len(_A_CHUNKS):
            ra.flush(i)
        if i < len(_B_CHUNKS):
            rb.flush(i)
    ra.finish()
    rb.finish()


def kernel(wpe):
    mesh = plsc.VectorSubcoreMesh(core_axis_name="c", subcore_axis_name="s")
    run = functools.partial(
        pl.kernel,
        mesh=mesh,
        out_type=jax.ShapeDtypeStruct((_BLOCK_SIZE, _N_EMBD), jnp.float32),
        scratch_types=(
            [
                pltpu.VMEM((_NBUF_A, _CHUNK, _N_EMBD), jnp.float32),
                pltpu.VMEM_SHARED((_NS, _NBUF_B, _CHUNK, _N_EMBD), jnp.float32),
            ]
            + [pltpu.SemaphoreType.DMA] * (2 * _NBUF_A + 2 * _NBUF_B)
        ),
    )(_sc_copy)
    return run(wpe)[None]


# speedup vs baseline: 1.0388x; 1.0057x over previous
import functools

import jax
import jax.numpy as jnp
from jax import lax
from jax.experimental import pallas as pl
from jax.experimental.pallas import tpu as pltpu
from jax.experimental.pallas import tpu_sc as plsc

_BLOCK_SIZE = 8192
_N_EMBD = 1024

_info = plsc.get_sparse_core_info()
_NC, _NS = _info.num_cores, _info.num_subcores
_NW = _NC * _NS
_ROWS_PER_W = _BLOCK_SIZE // _NW

_CHUNK = 32
_NBUF_A = 2
_NBUF_B = 2
_NCHUNKS = _ROWS_PER_W // _CHUNK
_NLOCAL = _NCHUNKS // 2


class _Ring:
    def __init__(self, in_fn, out_fn, n, nbuf):
        self.in_fn, self.out_fn, self.n, self.nbuf = in_fn, out_fn, n, nbuf
        self.ins = [None] * n
        self.outs = [None] * n

    def start(self, i):
        if i >= self.nbuf:
            self.outs[i - self.nbuf].wait()
        self.ins[i] = self.in_fn(i)

    def flush(self, i):
        if i >= 1:
            self.ins[i - 1].wait()
            self.outs[i - 1] = self.out_fn(i - 1)

    def finish(self):
        self.ins[self.n - 1].wait()
        self.outs[self.n - 1] = self.out_fn(self.n - 1)
        for j in range(max(0, self.n - self.nbuf), self.n):
            self.outs[j].wait()


def _sc_copy(wpe_hbm, out_hbm, vbuf, sbuf, *sems):
    via = sems[0:_NBUF_A]
    voa = sems[_NBUF_A : 2 * _NBUF_A]
    sib = sems[2 * _NBUF_A : 2 * _NBUF_A + _NBUF_B]
    sob = sems[2 * _NBUF_A + _NBUF_B :]
    cid = lax.axis_index("c")
    sid = lax.axis_index("s")
    base = (cid * _NS + sid) * _ROWS_PER_W

    def rows(g):
        return pl.ds(base + g * _CHUNK, _CHUNK)

    def a_in(i):
        return pltpu.async_copy(wpe_hbm.at[rows(2 * i)], vbuf.at[i % _NBUF_A], via[i % _NBUF_A])

    def a_out(i):
        return pltpu.async_copy(vbuf.at[i % _NBUF_A], out_hbm.at[rows(2 * i)], voa[i % _NBUF_A])

    def b_in(i):
        return pltpu.async_copy(
            wpe_hbm.at[rows(2 * i + 1)], sbuf.at[sid, i % _NBUF_B], sib[i % _NBUF_B]
        )

    def b_out(i):
        return pltpu.async_copy(
            sbuf.at[sid, i % _NBUF_B], out_hbm.at[rows(2 * i + 1)], sob[i % _NBUF_B]
        )

    ra = _Ring(a_in, a_out, _NLOCAL, _NBUF_A)
    rb = _Ring(b_in, b_out, _NLOCAL, _NBUF_B)
    for i in range(_NLOCAL):
        ra.start(i)
        rb.start(i)
        ra.flush(i)
        rb.flush(i)
    ra.finish()
    rb.finish()


def kernel(wpe):
    mesh = plsc.VectorSubcoreMesh(core_axis_name="c", subcore_axis_name="s")
    run = functools.partial(
        pl.kernel,
        mesh=mesh,
        out_type=jax.ShapeDtypeStruct((_BLOCK_SIZE, _N_EMBD), jnp.float32),
        scratch_types=(
            [
                pltpu.VMEM((_NBUF_A, _CHUNK, _N_EMBD), jnp.float32),
                pltpu.VMEM_SHARED((_NS, _NBUF_B, _CHUNK, _N_EMBD), jnp.float32),
            ]
            + [pltpu.SemaphoreType.DMA] * (2 * _NBUF_A + 2 * _NBUF_B)
        ),
    )(_sc_copy)
    return run(wpe)[None]
